# Initial kernel scaffold; baseline (speedup 1.0000x reference)
import jax
import jax.numpy as jnp
from jax.experimental import pallas as pl


def kernel(table_a_emb, table_b_emb, keys_a, keys_b, W1, b1, W2, b2):
    raise NotImplementedError("write your pallas kernel here")


# speedup vs baseline: 146.0208x; 146.0208x over previous
import jax
import jax.numpy as jnp
from jax.experimental import pallas as pl
from jax.experimental.pallas import tpu as pltpu

_D = 64
_NB = 100000
_NA = 1024
_BUCKETS = 1024
_CAP = 256
_CHUNK = 2000
_NCHUNK = _NB // _CHUNK
_SUB = 250
_G = _CHUNK // _SUB


def _seg_kernel(keys_ref, emb_ref, sum_out, cnt_out, acc, cnt):
    step = pl.program_id(0)

    @pl.when(step == 0)
    def _init():
        acc[...] = jnp.zeros((_BUCKETS, _D), jnp.float32)
        cnt[...] = jnp.zeros((_BUCKETS, 1), jnp.float32)

    keys = keys_ref[0]
    h = jnp.sum(keys, axis=1) % _BUCKETS
    iota_b = jax.lax.broadcasted_iota(jnp.int32, (_SUB, _BUCKETS), 1)
    ltri = jax.lax.broadcasted_iota(
        jnp.int32, (_SUB, _SUB), 0
    ) > jax.lax.broadcasted_iota(jnp.int32, (_SUB, _SUB), 1)
    emb = emb_ref[0]
    for g in range(_G):
        h_g = jax.lax.slice(h, (g * _SUB,), ((g + 1) * _SUB,))
        oh_g = (h_g[:, None] == iota_b).astype(jnp.float32)
        same = (h_g[:, None] == h_g[None, :]) & ltri
        rank = jnp.sum(same.astype(jnp.float32), axis=1)
        prev = jnp.dot(oh_g, cnt[...], preferred_element_type=jnp.float32)
        include = ((prev[:, 0] + rank) < _CAP).astype(jnp.float32)
        ow = oh_g * include[:, None]
        emb_g = jax.lax.slice(emb, (g * _SUB, 0), ((g + 1) * _SUB, _D))
        acc[...] += jax.lax.dot_general(
            ow, emb_g, (((0,), (0,)), ((), ())),
            preferred_element_type=jnp.float32,
        )
        cnt[...] += jnp.sum(oh_g, axis=0)[:, None]

    @pl.when(step == _NCHUNK - 1)
    def _fin():
        sum_out[...] = acc[...]
        cnt_out[...] = cnt[...]


def _mlp_kernel(ta_ref, ka_ref, sum_ref, cnt_ref, w1_ref, b1_ref, w2_ref, b2_ref, out_ref):
    ha = jnp.sum(ka_ref[...], axis=1) % _BUCKETS
    iota_b = jax.lax.broadcasted_iota(jnp.int32, (_NA, _BUCKETS), 1)
    oh = (ha[:, None] == iota_b).astype(jnp.float32)
    agg_sum = jnp.dot(oh, sum_ref[...], preferred_element_type=jnp.float32)
    cnt_a = jnp.dot(oh, cnt_ref[...], preferred_element_type=jnp.float32)
    scale = 1.0 / jnp.maximum(jnp.minimum(cnt_a, float(_CAP)), 1.0)
    agg = agg_sum * scale
    w1 = w1_ref[...]
    x = (
        jnp.dot(ta_ref[...], w1[:_D], preferred_element_type=jnp.float32)
        + jnp.dot(agg, w1[_D:], preferred_element_type=jnp.float32)
        + b1_ref[...]
    )
    x = jnp.maximum(x, 0.0)
    out_ref[...] = (
        jnp.dot(x, w2_ref[...], preferred_element_type=jnp.float32) + b2_ref[...]
    )


def kernel(table_a_emb, table_b_emb, keys_a, keys_b, W1, b1, W2, b2):
    keys_r = keys_b.reshape(_NCHUNK, _CHUNK, 4)
    emb_r = table_b_emb.reshape(_NCHUNK, _CHUNK, _D)
    bucket_sum, bucket_cnt = pl.pallas_call(
        _seg_kernel,
        grid=(_NCHUNK,),
        in_specs=[
            pl.BlockSpec((1, _CHUNK, 4), lambda i: (i, 0, 0)),
            pl.BlockSpec((1, _CHUNK, _D), lambda i: (i, 0, 0)),
        ],
        out_specs=[
            pl.BlockSpec((_BUCKETS, _D), lambda i: (0, 0)),
            pl.BlockSpec((_BUCKETS, 1), lambda i: (0, 0)),
        ],
        out_shape=[
            jax.ShapeDtypeStruct((_BUCKETS, _D), jnp.float32),
            jax.ShapeDtypeStruct((_BUCKETS, 1), jnp.float32),
        ],
        scratch_shapes=[
            pltpu.VMEM((_BUCKETS, _D), jnp.float32),
            pltpu.VMEM((_BUCKETS, 1), jnp.float32),
        ],
    )(keys_r, emb_r)

    out = pl.pallas_call(
        _mlp_kernel,
        out_shape=jax.ShapeDtypeStruct((_NA, _D), jnp.float32),
    )(
        table_a_emb,
        keys_a,
        bucket_sum,
        bucket_cnt,
        W1,
        b1.reshape(1, -1),
        W2,
        b2.reshape(1, -1),
    )
    return out
